# Initial kernel scaffold; baseline (speedup 1.0000x reference)
import jax
import jax.numpy as jnp
from jax.experimental import pallas as pl


def kernel(node_attrs, node_feats, edge_attrs, edge_feats, edge_index, W_up, W_r0, W_r1, W_r2, W_r3, W_lin, W_skip):
    raise NotImplementedError("write your pallas kernel here")


# speedup vs baseline: 2.1054x; 2.1054x over previous
import functools
import math

import jax
import jax.numpy as jnp
import numpy as np
from jax import lax
from jax.experimental import pallas as pl
from jax.experimental.pallas import tpu as pltpu
from jax.experimental.pallas import tpu_sc as plsc

_rng = np.random.RandomState(0)
_Z = _rng.randn(1000000)
_SILU_CST = float(1.0 / np.sqrt(np.mean((_Z / (1.0 + np.exp(-_Z))) ** 2)))

_N = 10000
_E = 320000
_D = 128
_K = 16
_R = 8
_HID = 64
_AVG = 32.0


def _silu_n(x):
    return _SILU_CST * x * jax.nn.sigmoid(x)


def _edge_w_body(ef_ref, ea_ref, w0_ref, w1_ref, w2_ref, w3_ref, o_ref):
    x = ef_ref[...]
    x = _silu_n(jnp.dot(x, w0_ref[...], preferred_element_type=jnp.float32)
                * (1.0 / math.sqrt(_R)))
    x = _silu_n(jnp.dot(x, w1_ref[...], preferred_element_type=jnp.float32)
                * (1.0 / math.sqrt(_HID)))
    x = _silu_n(jnp.dot(x, w2_ref[...], preferred_element_type=jnp.float32)
                * (1.0 / math.sqrt(_HID)))
    w = jnp.dot(x, w3_ref[...], preferred_element_type=jnp.float32) * (
        1.0 / math.sqrt(_HID))
    o_ref[...] = w * ea_ref[...]


def _edge_w(edge_feats, edge_attrs, W_r0, W_r1, W_r2, W_r3):
    BE = 4000
    grid = _E // BE
    return pl.pallas_call(
        _edge_w_body,
        grid=(grid,),
        in_specs=[
            pl.BlockSpec((BE, _R), lambda i: (i, 0)),
            pl.BlockSpec((BE, 1), lambda i: (i, 0)),
            pl.BlockSpec((_R, _HID), lambda i: (0, 0)),
            pl.BlockSpec((_HID, _HID), lambda i: (0, 0)),
            pl.BlockSpec((_HID, _HID), lambda i: (0, 0)),
            pl.BlockSpec((_HID, _D), lambda i: (0, 0)),
        ],
        out_specs=pl.BlockSpec((BE, _D), lambda i: (i, 0)),
        out_shape=jax.ShapeDtypeStruct((_E, _D), jnp.float32),
    )(edge_feats, edge_attrs, W_r0, W_r1, W_r2, W_r3)


def _linear_up_body(nf_ref, wu_ref, o_ref):
    o_ref[...] = jnp.dot(nf_ref[...], wu_ref[...],
                         preferred_element_type=jnp.float32) * (
        1.0 / math.sqrt(_D))


def _linear_up(node_feats, W_up):
    return pl.pallas_call(
        _linear_up_body,
        out_shape=jax.ShapeDtypeStruct((_N, _D), jnp.float32),
    )(node_feats, W_up)


def _skip_body(nf_ref, na_ref, ws_ref, o_ref):
    nf = nf_ref[...]
    acc = jnp.zeros((nf.shape[0], _D), jnp.float32)
    for v in range(_K):
        acc += na_ref[:, v:v + 1] * jnp.dot(
            nf, ws_ref[v], preferred_element_type=jnp.float32)
    o_ref[...] = acc * (1.0 / math.sqrt(float(_D * _K)))


def _skip(node_feats, node_attrs, W_skip_t):
    BN = 2000
    grid = _N // BN
    return pl.pallas_call(
        _skip_body,
        grid=(grid,),
        in_specs=[
            pl.BlockSpec((BN, _D), lambda i: (i, 0)),
            pl.BlockSpec((BN, _K), lambda i: (i, 0)),
            pl.BlockSpec((_K, _D, _D), lambda i: (0, 0, 0)),
        ],
        out_specs=pl.BlockSpec((BN, _D), lambda i: (i, 0)),
        out_shape=jax.ShapeDtypeStruct((_N, _D), jnp.float32),
    )(node_feats, node_attrs, W_skip_t)


def _final_body(p_ref, wl_ref, o_ref):
    m = p_ref[0] + p_ref[1]
    o_ref[...] = jnp.dot(m, wl_ref[...], preferred_element_type=jnp.float32) * (
        1.0 / (math.sqrt(_D) * _AVG))


def _final(parts, W_lin):
    return pl.pallas_call(
        _final_body,
        out_shape=jax.ShapeDtypeStruct((_N, _D), jnp.float32),
    )(parts, W_lin)


_NTILES = 32
_EPT = _E // _NTILES
_C = 80
_STEPS = _EPT // _C
_STRIPE = 1000
_ZROWS = 200


def _sc_body(h_hbm, w_hbm, send_hbm, recv_hbm, out_hbm,
             sidx, ridx, wv, rows, zv, acc, sem):
    cid = lax.axis_index("c")
    sid = lax.axis_index("s")
    wid = sid * 2 + cid
    base = wid * _EPT

    def zrow(r, carry):
        for j in range(8):
            zv[r, pl.ds(j * 16, 16)] = jnp.zeros((16,), jnp.float32)
        return carry

    lax.fori_loop(0, _ZROWS, zrow, 0)

    @pl.when(sid < 10)
    def _zero():
        for j in range(_STRIPE // _ZROWS):
            pltpu.sync_copy(
                zv, acc.at[pl.ds(sid * _STRIPE + j * _ZROWS, _ZROWS)])

    plsc.subcore_barrier()

    def step(it, carry):
        off = base + it * _C
        pltpu.sync_copy(send_hbm.at[pl.ds(off, _C)], sidx)
        pltpu.sync_copy(recv_hbm.at[pl.ds(off, _C)], ridx)
        pltpu.sync_copy(w_hbm.at[pl.ds(off, _C)], wv)
        pltpu.async_copy(h_hbm.at[sidx], rows, sem).wait()

        def mrow(r, c2):
            for j in range(8):
                sl = pl.ds(j * 16, 16)
                rows[r, sl] = rows[r, sl] * wv[r, sl]
            return c2

        lax.fori_loop(0, _C, mrow, 0)
        pltpu.sync_copy(rows, acc.at[ridx], add=True)
        return carry

    lax.fori_loop(0, _STEPS, step, 0)
    plsc.subcore_barrier()

    @pl.when(sid < 10)
    def _flush():
        pltpu.sync_copy(acc.at[pl.ds(sid * _STRIPE, _STRIPE)],
                        out_hbm.at[cid, pl.ds(sid * _STRIPE, _STRIPE)])


def _sc_messages(h, w, send, recv):
    mesh = plsc.VectorSubcoreMesh(core_axis_name="c", subcore_axis_name="s")
    fn = functools.partial(
        pl.kernel,
        mesh=mesh,
        out_type=jax.ShapeDtypeStruct((2, _N, _D), jnp.float32),
        scratch_types=[
            pltpu.VMEM((_C,), jnp.int32),
            pltpu.VMEM((_C,), jnp.int32),
            pltpu.VMEM((_C, _D), jnp.float32),
            pltpu.VMEM((_C, _D), jnp.float32),
            pltpu.VMEM((_ZROWS, _D), jnp.float32),
            pltpu.VMEM_SHARED((_N, _D), jnp.float32),
            pltpu.SemaphoreType.DMA,
        ],
    )(_sc_body)
    return fn(h, w, send, recv)


def kernel(node_attrs, node_feats, edge_attrs, edge_feats, edge_index,
           W_up, W_r0, W_r1, W_r2, W_r3, W_lin, W_skip):
    send = edge_index[0]
    recv = edge_index[1]
    w = _edge_w(edge_feats, edge_attrs, W_r0, W_r1, W_r2, W_r3)
    h = _linear_up(node_feats, W_up)
    sc = _skip(node_feats, node_attrs, jnp.transpose(W_skip, (1, 0, 2)))
    parts = _sc_messages(h, w, send, recv)
    message = _final(parts, W_lin)
    return (message[:, :, None], sc)


# speedup vs baseline: 2.7175x; 1.2907x over previous
import functools
import math

import jax
import jax.numpy as jnp
import numpy as np
from jax import lax
from jax.experimental import pallas as pl
from jax.experimental.pallas import tpu as pltpu
from jax.experimental.pallas import tpu_sc as plsc

_rng = np.random.RandomState(0)
_Z = _rng.randn(1000000)
_SILU_CST = float(1.0 / np.sqrt(np.mean((_Z / (1.0 + np.exp(-_Z))) ** 2)))

_N = 10000
_E = 320000
_D = 128
_K = 16
_R = 8
_HID = 64
_AVG = 32.0


def _silu_n(x):
    return _SILU_CST * x * jax.nn.sigmoid(x)


def _edge_w_body(ef_ref, ea_ref, w0_ref, w1_ref, w2_ref, w3_ref, o_ref):
    x = ef_ref[...]
    x = _silu_n(jnp.dot(x, w0_ref[...], preferred_element_type=jnp.float32)
                * (1.0 / math.sqrt(_R)))
    x = _silu_n(jnp.dot(x, w1_ref[...], preferred_element_type=jnp.float32)
                * (1.0 / math.sqrt(_HID)))
    x = _silu_n(jnp.dot(x, w2_ref[...], preferred_element_type=jnp.float32)
                * (1.0 / math.sqrt(_HID)))
    w = jnp.dot(x, w3_ref[...], preferred_element_type=jnp.float32) * (
        1.0 / math.sqrt(_HID))
    o_ref[...] = w * ea_ref[...]


def _edge_w(edge_feats, edge_attrs, W_r0, W_r1, W_r2, W_r3):
    BE = 4000
    grid = _E // BE
    return pl.pallas_call(
        _edge_w_body,
        grid=(grid,),
        in_specs=[
            pl.BlockSpec((BE, _R), lambda i: (i, 0)),
            pl.BlockSpec((BE, 1), lambda i: (i, 0)),
            pl.BlockSpec((_R, _HID), lambda i: (0, 0)),
            pl.BlockSpec((_HID, _HID), lambda i: (0, 0)),
            pl.BlockSpec((_HID, _HID), lambda i: (0, 0)),
            pl.BlockSpec((_HID, _D), lambda i: (0, 0)),
        ],
        out_specs=pl.BlockSpec((BE, _D), lambda i: (i, 0)),
        out_shape=jax.ShapeDtypeStruct((_E, _D), jnp.float32),
    )(edge_feats, edge_attrs, W_r0, W_r1, W_r2, W_r3)


def _linear_up_body(nf_ref, wu_ref, o_ref):
    o_ref[...] = jnp.dot(nf_ref[...], wu_ref[...],
                         preferred_element_type=jnp.float32) * (
        1.0 / math.sqrt(_D))


def _linear_up(node_feats, W_up):
    return pl.pallas_call(
        _linear_up_body,
        out_shape=jax.ShapeDtypeStruct((_N, _D), jnp.float32),
    )(node_feats, W_up)


def _skip_body(nf_ref, na_ref, ws_ref, o_ref):
    nf = nf_ref[...]
    acc = jnp.zeros((nf.shape[0], _D), jnp.float32)
    for v in range(_K):
        acc += na_ref[:, v:v + 1] * jnp.dot(
            nf, ws_ref[v], preferred_element_type=jnp.float32)
    o_ref[...] = acc * (1.0 / math.sqrt(float(_D * _K)))


def _skip(node_feats, node_attrs, W_skip_t):
    BN = 2000
    grid = _N // BN
    return pl.pallas_call(
        _skip_body,
        grid=(grid,),
        in_specs=[
            pl.BlockSpec((BN, _D), lambda i: (i, 0)),
            pl.BlockSpec((BN, _K), lambda i: (i, 0)),
            pl.BlockSpec((_K, _D, _D), lambda i: (0, 0, 0)),
        ],
        out_specs=pl.BlockSpec((BN, _D), lambda i: (i, 0)),
        out_shape=jax.ShapeDtypeStruct((_N, _D), jnp.float32),
    )(node_feats, node_attrs, W_skip_t)


def _final_body(p_ref, wl_ref, o_ref):
    m = p_ref[0] + p_ref[1]
    o_ref[...] = jnp.dot(m, wl_ref[...], preferred_element_type=jnp.float32) * (
        1.0 / (math.sqrt(_D) * _AVG))


def _final(parts, W_lin):
    return pl.pallas_call(
        _final_body,
        out_shape=jax.ShapeDtypeStruct((_N, _D), jnp.float32),
    )(parts, W_lin)


_NTILES = 32
_EPT = _E // _NTILES
_C = 40
_STEPS = _EPT // _C
_STRIPE = 1000
_ZROWS = 8


def _sc_body(h_hbm, w_hbm, idx_hbm, out_hbm,
             ibuf, wv, rows, zv, acc,
             semi0, semi1, semw0, semw1, semg0, semg1):
    cid = lax.axis_index("c")
    sid = lax.axis_index("s")
    wid = sid * 2 + cid
    base = wid * _EPT

    def i_copy(i, slot, sem):
        return pltpu.make_async_copy(idx_hbm.at[wid, i], ibuf.at[slot], sem)

    def w_copy(i, slot, sem):
        return pltpu.make_async_copy(
            w_hbm.at[pl.ds(base + i * _C, _C)], wv.at[slot], sem)

    def g_copy(slot, sem):
        return pltpu.make_async_copy(
            h_hbm.at[ibuf.at[slot, 0]], rows.at[slot], sem)

    def mult(slot):
        def mrow(r, c2):
            for j in range(8):
                sl = pl.ds(j * 16, 16)
                rows[slot, r, sl] = rows[slot, r, sl] * wv[slot, r, sl]
            return c2

        lax.fori_loop(0, _C, mrow, 0)

    i_copy(0, 0, semi0).start()
    i_copy(1, 1, semi1).start()
    w_copy(0, 0, semw0).start()
    w_copy(1, 1, semw1).start()
    i_copy(0, 0, semi0).wait()
    g_copy(0, semg0).start()

    def zrow(r, carry):
        for j in range(8):
            zv[r, pl.ds(j * 16, 16)] = jnp.zeros((16,), jnp.float32)
        return carry

    lax.fori_loop(0, _ZROWS, zrow, 0)

    @pl.when(sid < 10)
    def _zero():
        for j in range(_STRIPE // _ZROWS):
            pltpu.sync_copy(
                zv, acc.at[pl.ds(sid * _STRIPE + j * _ZROWS, _ZROWS)])

    plsc.subcore_barrier()

    def body(k, carry):
        i0 = 2 * k
        i1 = i0 + 1
        i_copy(i1, 1, semi1).wait()
        g_copy(1, semg1).start()
        w_copy(i0, 0, semw0).wait()
        g_copy(0, semg0).wait()
        mult(0)
        pltpu.sync_copy(rows.at[0], acc.at[ibuf.at[0, 1]], add=True)

        @pl.when(k < _STEPS // 2 - 1)
        def _pre0():
            i_copy(i0 + 2, 0, semi0).start()
            w_copy(i0 + 2, 0, semw0).start()
            i_copy(i0 + 2, 0, semi0).wait()
            g_copy(0, semg0).start()

        w_copy(i1, 1, semw1).wait()
        g_copy(1, semg1).wait()
        mult(1)
        pltpu.sync_copy(rows.at[1], acc.at[ibuf.at[1, 1]], add=True)

        @pl.when(k < _STEPS // 2 - 1)
        def _pre1():
            i_copy(i1 + 2, 1, semi1).start()
            w_copy(i1 + 2, 1, semw1).start()

        return carry

    lax.fori_loop(0, _STEPS // 2, body, 0)
    plsc.subcore_barrier()

    @pl.when(sid < 10)
    def _flush():
        pltpu.sync_copy(acc.at[pl.ds(sid * _STRIPE, _STRIPE)],
                        out_hbm.at[cid, pl.ds(sid * _STRIPE, _STRIPE)])


def _sc_messages(h, w, send, recv):
    mesh = plsc.VectorSubcoreMesh(core_axis_name="c", subcore_axis_name="s")
    fn = functools.partial(
        pl.kernel,
        mesh=mesh,
        out_type=jax.ShapeDtypeStruct((2, _N, _D), jnp.float32),
        scratch_types=[
            pltpu.VMEM((2, 2, _C), jnp.int32),
            pltpu.VMEM((2, _C, _D), jnp.float32),
            pltpu.VMEM((2, _C, _D), jnp.float32),
            pltpu.VMEM((_ZROWS, _D), jnp.float32),
            pltpu.VMEM_SHARED((_N, _D), jnp.float32),
            pltpu.SemaphoreType.DMA,
            pltpu.SemaphoreType.DMA,
            pltpu.SemaphoreType.DMA,
            pltpu.SemaphoreType.DMA,
            pltpu.SemaphoreType.DMA,
            pltpu.SemaphoreType.DMA,
        ],
    )(_sc_body)
    idx4 = jnp.stack([send, recv]).reshape(
        2, _NTILES, _STEPS, _C).transpose(1, 2, 0, 3)
    return fn(h, w, idx4)


def kernel(node_attrs, node_feats, edge_attrs, edge_feats, edge_index,
           W_up, W_r0, W_r1, W_r2, W_r3, W_lin, W_skip):
    send = edge_index[0]
    recv = edge_index[1]
    w = _edge_w(edge_feats, edge_attrs, W_r0, W_r1, W_r2, W_r3)
    h = _linear_up(node_feats, W_up)
    sc = _skip(node_feats, node_attrs, jnp.transpose(W_skip, (1, 0, 2)))
    parts = _sc_messages(h, w, send, recv)
    message = _final(parts, W_lin)
    return (message[:, :, None], sc)


# speedup vs baseline: 3.0956x; 1.1391x over previous
import functools
import math

import jax
import jax.numpy as jnp
import numpy as np
from jax import lax
from jax.experimental import pallas as pl
from jax.experimental.pallas import tpu as pltpu
from jax.experimental.pallas import tpu_sc as plsc

_rng = np.random.RandomState(0)
_Z = _rng.randn(1000000)
_SILU_CST = float(1.0 / np.sqrt(np.mean((_Z / (1.0 + np.exp(-_Z))) ** 2)))

_N = 10000
_E = 320000
_D = 128
_K = 16
_R = 8
_HID = 64
_AVG = 32.0


def _silu_n(x):
    return _SILU_CST * x * (0.5 * (1.0 + jnp.tanh(0.5 * x)))


def _edge_w_body(ef_ref, ea_ref, w0_ref, w1_ref, w2_ref, w3_ref, o_ref):
    x = ef_ref[...]
    x = _silu_n(jnp.dot(x, w0_ref[...], preferred_element_type=jnp.float32)
                * (1.0 / math.sqrt(_R)))
    x = _silu_n(jnp.dot(x, w1_ref[...], preferred_element_type=jnp.float32)
                * (1.0 / math.sqrt(_HID)))
    x = _silu_n(jnp.dot(x, w2_ref[...], preferred_element_type=jnp.float32)
                * (1.0 / math.sqrt(_HID)))
    w = jnp.dot(x, w3_ref[...], preferred_element_type=jnp.float32) * (
        1.0 / math.sqrt(_HID))
    wbf = (w * ea_ref[...]).astype(jnp.bfloat16)
    lo = jax.lax.bitcast_convert_type(
        wbf[:, :_D // 2], jnp.uint16).astype(jnp.uint32)
    hi = jax.lax.bitcast_convert_type(
        wbf[:, _D // 2:], jnp.uint16).astype(jnp.uint32)
    o_ref[...] = jax.lax.bitcast_convert_type((hi << 16) | lo, jnp.int32)


def _edge_w(edge_feats, edge_attrs, W_r0, W_r1, W_r2, W_r3):
    BE = 4000
    grid = _E // BE
    return pl.pallas_call(
        _edge_w_body,
        grid=(grid,),
        in_specs=[
            pl.BlockSpec((BE, _R), lambda i: (i, 0)),
            pl.BlockSpec((BE, 1), lambda i: (i, 0)),
            pl.BlockSpec((_R, _HID), lambda i: (0, 0)),
            pl.BlockSpec((_HID, _HID), lambda i: (0, 0)),
            pl.BlockSpec((_HID, _HID), lambda i: (0, 0)),
            pl.BlockSpec((_HID, _D), lambda i: (0, 0)),
        ],
        out_specs=pl.BlockSpec((BE, _D // 2), lambda i: (i, 0)),
        out_shape=jax.ShapeDtypeStruct((_E, _D // 2), jnp.int32),
    )(edge_feats, edge_attrs, W_r0, W_r1, W_r2, W_r3)


def _linear_up_body(nf_ref, wu_ref, o_ref):
    o_ref[...] = jnp.dot(nf_ref[...], wu_ref[...],
                         preferred_element_type=jnp.float32) * (
        1.0 / math.sqrt(_D))


def _linear_up(node_feats, W_up):
    return pl.pallas_call(
        _linear_up_body,
        out_shape=jax.ShapeDtypeStruct((_N, _D), jnp.float32),
    )(node_feats, W_up)


def _skip_body(nf_ref, na_ref, ws_ref, o_ref):
    nf = nf_ref[...]
    acc = jnp.zeros((nf.shape[0], _D), jnp.float32)
    for v in range(_K):
        acc += na_ref[:, v:v + 1] * jnp.dot(
            nf, ws_ref[v], preferred_element_type=jnp.float32)
    o_ref[...] = acc * (1.0 / math.sqrt(float(_D * _K)))


def _skip(node_feats, node_attrs, W_skip_t):
    BN = 2000
    grid = _N // BN
    return pl.pallas_call(
        _skip_body,
        grid=(grid,),
        in_specs=[
            pl.BlockSpec((BN, _D), lambda i: (i, 0)),
            pl.BlockSpec((BN, _K), lambda i: (i, 0)),
            pl.BlockSpec((_K, _D, _D), lambda i: (0, 0, 0)),
        ],
        out_specs=pl.BlockSpec((BN, _D), lambda i: (i, 0)),
        out_shape=jax.ShapeDtypeStruct((_N, _D), jnp.float32),
    )(node_feats, node_attrs, W_skip_t)


def _final_body(p_ref, wl_ref, o_ref):
    m = p_ref[0] + p_ref[1]
    o_ref[...] = jnp.dot(m, wl_ref[...], preferred_element_type=jnp.float32) * (
        1.0 / (math.sqrt(_D) * _AVG))


def _final(parts, W_lin):
    return pl.pallas_call(
        _final_body,
        out_shape=jax.ShapeDtypeStruct((_N, _D), jnp.float32),
    )(parts, W_lin)


_NTILES = 32
_EPT = _E // _NTILES
_C = 80
_STEPS = _EPT // _C
_STRIPE = 1000
_ZROWS = 8

def _sc_body(h_hbm, w_hbm, idx_hbm, out_hbm,
             ibuf, wv, rows, zv, acc,
             semi0, semi1, semw0, semw1, semg0, semg1):
    cid = lax.axis_index("c")
    sid = lax.axis_index("s")
    wid = sid * 2 + cid
    base = wid * _EPT

    def i_copy(i, slot, sem):
        return pltpu.make_async_copy(idx_hbm.at[wid, i], ibuf.at[slot], sem)

    def w_copy(i, slot, sem):
        return pltpu.make_async_copy(
            w_hbm.at[pl.ds(base + i * _C, _C)], wv.at[slot], sem)

    def scat(slot):
        pltpu.sync_copy(rows.at[slot], acc.at[ibuf.at[slot, 1]], add=True)

    def g_copy(slot, sem):
        return pltpu.make_async_copy(
            h_hbm.at[ibuf.at[slot, 0]], rows.at[slot], sem)

    def mult(slot):
        def mrow(r, c2):
            for j in range(4):
                u = wv[slot, r, pl.ds(j * 16, 16)]
                wlo = jax.lax.bitcast_convert_type(u << 16, jnp.float32)
                whi = jax.lax.bitcast_convert_type(u & jnp.int32(-65536), jnp.float32)
                lo = pl.ds(j * 16, 16)
                hi = pl.ds(_D // 2 + j * 16, 16)
                rows[slot, r, lo] = rows[slot, r, lo] * wlo
                rows[slot, r, hi] = rows[slot, r, hi] * whi
            return c2

        lax.fori_loop(0, _C, mrow, 0)

    i_copy(0, 0, semi0).start()
    i_copy(1, 1, semi1).start()
    w_copy(0, 0, semw0).start()
    w_copy(1, 1, semw1).start()
    i_copy(0, 0, semi0).wait()
    g_copy(0, semg0).start()

    def zrow(r, carry):
        for j in range(8):
            zv[r, pl.ds(j * 16, 16)] = jnp.zeros((16,), jnp.float32)
        return carry

    lax.fori_loop(0, _ZROWS, zrow, 0)

    @pl.when(sid < 10)
    def _zero():
        for j in range(_STRIPE // _ZROWS):
            pltpu.sync_copy(
                zv, acc.at[pl.ds(sid * _STRIPE + j * _ZROWS, _ZROWS)])

    plsc.subcore_barrier()

    def body(k, carry):
        i0 = 2 * k
        i1 = i0 + 1
        i_copy(i1, 1, semi1).wait()
        g_copy(1, semg1).start()
        w_copy(i0, 0, semw0).wait()
        g_copy(0, semg0).wait()
        mult(0)
        scat(0)
        i_copy(i0 + 2, 0, semi0).start()
        w_copy(i0 + 2, 0, semw0).start()
        i_copy(i0 + 2, 0, semi0).wait()
        g_copy(0, semg0).start()

        w_copy(i1, 1, semw1).wait()
        g_copy(1, semg1).wait()
        mult(1)
        scat(1)

        @pl.when(k < _STEPS // 2 - 1)
        def _pre1():
            i_copy(i1 + 2, 1, semi1).start()
            w_copy(i1 + 2, 1, semw1).start()

        return carry

    lax.fori_loop(0, _STEPS // 2, body, 0)
    w_copy(_STEPS - 1, 0, semw0).wait()
    g_copy(0, semg0).wait()
    mult(0)
    scat(0)
    plsc.subcore_barrier()

    @pl.when(sid < 10)
    def _flush():
        pltpu.sync_copy(acc.at[pl.ds(sid * _STRIPE, _STRIPE)],
                        out_hbm.at[cid, pl.ds(sid * _STRIPE, _STRIPE)])


def _sc_messages(h, w, send, recv):
    mesh = plsc.VectorSubcoreMesh(core_axis_name="c", subcore_axis_name="s")
    fn = functools.partial(
        pl.kernel,
        mesh=mesh,
        out_type=jax.ShapeDtypeStruct((2, _N, _D), jnp.float32),
        scratch_types=[
            pltpu.VMEM((2, 2, _C), jnp.int32),
            pltpu.VMEM((2, _C, _D // 2), jnp.int32),
            pltpu.VMEM((2, _C, _D), jnp.float32),
            pltpu.VMEM((_ZROWS, _D), jnp.float32),
            pltpu.VMEM_SHARED((_N, _D), jnp.float32),
            pltpu.SemaphoreType.DMA,
            pltpu.SemaphoreType.DMA,
            pltpu.SemaphoreType.DMA,
            pltpu.SemaphoreType.DMA,
            pltpu.SemaphoreType.DMA,
            pltpu.SemaphoreType.DMA,
        ],
    )(_sc_body)
    idx4 = jnp.stack([send, recv]).reshape(
        2, _NTILES, _STEPS, _C).transpose(1, 2, 0, 3)
    return fn(h, w, idx4)


def kernel(node_attrs, node_feats, edge_attrs, edge_feats, edge_index,
           W_up, W_r0, W_r1, W_r2, W_r3, W_lin, W_skip):
    send = edge_index[0]
    recv = edge_index[1]
    w = _edge_w(edge_feats, edge_attrs, W_r0, W_r1, W_r2, W_r3)
    h = _linear_up(node_feats, W_up)
    sc = _skip(node_feats, node_attrs, jnp.transpose(W_skip, (1, 0, 2)))
    parts = _sc_messages(h, w, send, recv)
    message = _final(parts, W_lin)
    return (message[:, :, None], sc)
